# baseline (speedup 1.0000x reference)
import functools

import jax
import jax.numpy as jnp
from jax import lax
from jax.experimental import pallas as pl
from jax.experimental.pallas import tpu as pltpu
from jax.experimental.pallas import tpu_sc as plsc

R = 512
N = 100000
B = 4096
L = 16
NC = 2
NS = 16
NW = NC * NS
RPW = R // NW
BP = B + 128

_HBM = pl.BlockSpec(memory_space=pltpu.MemorySpace.HBM)

CH = 256000
NCHUNKS = (R * N) // CH
SLOTS = 8
LAG = 4


def _copy_body(t_ref, o_ref, buf, sem_in, sem_out):
    def _in(i, p):
        return pltpu.make_async_copy(
            t_ref.at[pl.ds(i * CH, CH)], buf.at[p], sem_in.at[p])

    def _out(i, p):
        return pltpu.make_async_copy(
            buf.at[p], o_ref.at[pl.ds(i * CH, CH)], sem_out.at[p])

    for i in range(NCHUNKS + LAG):
        if i < NCHUNKS:
            p = i % SLOTS
            if i >= SLOTS:
                _out(i - SLOTS, p).wait()
            _in(i, p).start()
        j = i - LAG
        if 0 <= j < NCHUNKS:
            pj = j % SLOTS
            _in(j, pj).wait()
            _out(j, pj).start()
    for j in range(NCHUNKS - SLOTS, NCHUNKS):
        _out(j, j % SLOTS).wait()


_mesh = plsc.VectorSubcoreMesh(
    core_axis_name="c", subcore_axis_name="s", num_cores=NC, num_subcores=NS)


@functools.partial(
    pl.kernel,
    mesh=_mesh,
    compiler_params=pltpu.CompilerParams(needs_layout_passes=False),
    scratch_types=[
        pltpu.VMEM((B,), jnp.int32),
        pltpu.VMEM((N,), jnp.int32),
        pltpu.VMEM((BP,), jnp.int32),
        pltpu.VMEM((BP,), jnp.int32),
        pltpu.VMEM((B,), jnp.float32),
        pltpu.VMEM((B,), jnp.float32),
        pltpu.VMEM((B // 128, 128), jnp.int32),
        pltpu.VMEM((B // 128, 128), jnp.float32),
        pltpu.SemaphoreType.DMA,
        pltpu.SemaphoreType.DMA,
    ],
)
def _sc_scatter(idx_hbm, v_hbm, out_hbm,
                idx_v, m_v, cols_s, jw_s, vrow_a, vrow_b,
                addr_v, val_v, sem_v, sem_s):
    vrows = (vrow_a, vrow_b)
    wid = lax.axis_index("s") * NC + lax.axis_index("c")
    pltpu.sync_copy(idx_hbm, idx_v)
    lane = lax.iota(jnp.int32, L)
    neg1 = jnp.full((L,), -1, jnp.int32)

    def _init(k, _):
        m_v[pl.ds(k * L, L)] = neg1
        return 0

    lax.fori_loop(0, N // L, _init, 0)

    def _mb(k, _):
        idxc = idx_v[pl.ds(k * L, L)]
        jvec = k * L + lane
        plsc.store_scatter(m_v, [idxc], jvec)

        def _fix(i, __):
            w = plsc.load_gather(m_v, [idxc])
            plsc.store_scatter(m_v, [idxc], jvec, mask=jvec > w)
            return 0

        lax.fori_loop(0, 15, _fix, 0)
        return 0

    lax.fori_loop(0, B // L, _mb, 0)

    def _scan(k, off):
        mv = m_v[pl.ds(k * L, L)]
        valid = mv >= 0
        cols = k * L + lane
        plsc.store_compressed(cols_s.at[pl.ds(off, L)], cols, mask=valid)
        plsc.store_compressed(jw_s.at[pl.ds(off, L)], mv, mask=valid)
        return off + jnp.sum(valid.astype(jnp.int32))

    kcount = lax.fori_loop(0, N // L, _scan, jnp.int32(0))

    c0 = jnp.broadcast_to(cols_s[pl.ds(0, L)][0], (L,))
    j0 = jnp.broadcast_to(jw_s[pl.ds(0, L)][0], (L,))
    for i in range(128 // L):
        cols_s[pl.ds(kcount + i * L, L)] = c0
        jw_s[pl.ds(kcount + i * L, L)] = j0

    nch = (kcount + 127) // 128
    nsub = (kcount + L - 1) // L

    def _prefetch(r, buf):
        return pltpu.make_async_copy(v_hbm.at[wid * RPW + r], buf, sem_v)

    _prefetch(0, vrow_a).start()

    def _rowpair(g, _):
        for pp in range(2):
            r = g * 2 + pp
            buf = vrows[pp]
            _prefetch(r, buf).wait()

            @pl.when(r + 1 < RPW)
            def _():
                _prefetch(r + 1, vrows[1 - pp]).start()

            base = (wid * RPW + r) * N

            def _sub(k, __, buf=buf):
                s = k * L
                cc = cols_s[pl.ds(s, L)]
                addr_v[k // 8, pl.ds((k % 8) * L, L)] = cc + base
                w = jw_s[pl.ds(s, L)]
                val_v[k // 8, pl.ds((k % 8) * L, L)] = plsc.load_gather(
                    buf, [w])
                return 0

            lax.fori_loop(0, nsub, _sub, 0)

            for d in range(B // 128):
                @pl.when(d < nch)
                def _(d=d):
                    pltpu.async_copy(
                        val_v.at[d], out_hbm.at[addr_v.at[d]], sem_s)

            for d in range(B // 128):
                @pl.when(d < nch)
                def _():
                    pltpu.make_async_copy(
                        out_hbm.at[pl.ds(0, 128)], val_v.at[0], sem_s).wait()
        return 0

    lax.fori_loop(0, RPW // 2, _rowpair, 0)


def kernel(t, idx, v):
    t1 = t.reshape(R * N)
    v2 = v.reshape(R, B)

    out0 = pl.pallas_call(
        _copy_body,
        out_shape=jax.ShapeDtypeStruct((R * N,), jnp.float32),
        in_specs=[_HBM],
        out_specs=_HBM,
        scratch_shapes=[
            pltpu.VMEM((SLOTS, CH), jnp.float32),
            pltpu.SemaphoreType.DMA((SLOTS,)),
            pltpu.SemaphoreType.DMA((SLOTS,)),
        ],
    )(t1)

    ref = jax.new_ref(out0)
    _sc_scatter(idx, v2, ref)
    return ref[...].reshape(t.shape)


# speedup vs baseline: 1.7776x; 1.7776x over previous
import functools

import jax
import jax.numpy as jnp
from jax import lax
from jax.experimental import pallas as pl
from jax.experimental.pallas import tpu as pltpu
from jax.experimental.pallas import tpu_sc as plsc

R = 512
N = 100000
B = 4096
L = 16
NC = 2
NS = 16
NW = NC * NS
RPW = R // NW
BP = B + 128

_HBM = pl.BlockSpec(memory_space=pltpu.MemorySpace.HBM)

CH = 256000
NCHUNKS = (R * N) // CH
SLOTS = 8
LAG = 4


def _copy_body(t_ref, o_ref, buf, sem_in, sem_out):
    def _in(i, p):
        return pltpu.make_async_copy(
            t_ref.at[pl.ds(i * CH, CH)], buf.at[p], sem_in.at[p])

    def _out(i, p):
        return pltpu.make_async_copy(
            buf.at[p], o_ref.at[pl.ds(i * CH, CH)], sem_out.at[p])

    for i in range(NCHUNKS + LAG):
        if i < NCHUNKS:
            p = i % SLOTS
            if i >= SLOTS:
                _out(i - SLOTS, p).wait()
            _in(i, p).start()
        j = i - LAG
        if 0 <= j < NCHUNKS:
            pj = j % SLOTS
            _in(j, pj).wait()
            _out(j, pj).start()
    for j in range(NCHUNKS - SLOTS, NCHUNKS):
        _out(j, j % SLOTS).wait()


_mesh = plsc.VectorSubcoreMesh(
    core_axis_name="c", subcore_axis_name="s", num_cores=NC, num_subcores=NS)


@functools.partial(
    pl.kernel,
    mesh=_mesh,
    compiler_params=pltpu.CompilerParams(needs_layout_passes=False),
    scratch_types=[
        pltpu.VMEM((B,), jnp.int32),
        pltpu.VMEM((N,), jnp.int32),
        pltpu.VMEM((BP,), jnp.int32),
        pltpu.VMEM((BP,), jnp.int32),
        pltpu.VMEM((B,), jnp.float32),
        pltpu.VMEM((B,), jnp.float32),
        pltpu.VMEM((B // 128, 128), jnp.int32),
        pltpu.VMEM((B // 128, 128), jnp.float32),
        pltpu.SemaphoreType.DMA,
        pltpu.SemaphoreType.DMA,
    ],
)
def _sc_scatter(idx_hbm, v_hbm, out_hbm,
                idx_v, m_v, cols_s, jw_s, vrow_a, vrow_b,
                addr_v, val_v, sem_v, sem_s):
    vrows = (vrow_a, vrow_b)
    wid = lax.axis_index("s") * NC + lax.axis_index("c")
    pltpu.sync_copy(idx_hbm, idx_v)
    lane = lax.iota(jnp.int32, L)
    neg1 = jnp.full((L,), -1, jnp.int32)

    def _init(k, _):
        m_v[pl.ds(k * L, L)] = neg1
        return 0

    lax.fori_loop(0, N // L, _init, 0)

    def _mb(k, _):
        idxc = idx_v[pl.ds(k * L, L)]
        jvec = k * L + lane
        plsc.store_scatter(m_v, [idxc], jvec)

        def _fix(i, __):
            w = plsc.load_gather(m_v, [idxc])
            plsc.store_scatter(m_v, [idxc], jvec, mask=jvec > w)
            return 0

        lax.fori_loop(0, 15, _fix, 0)
        return 0

    lax.fori_loop(0, B // L, _mb, 0)

    def _scan(k, off):
        mv = m_v[pl.ds(k * L, L)]
        valid = mv >= 0
        cols = k * L + lane
        plsc.store_compressed(cols_s.at[pl.ds(off, L)], cols, mask=valid)
        plsc.store_compressed(jw_s.at[pl.ds(off, L)], mv, mask=valid)
        return off + plsc.all_reduce_population_count(valid)[0]

    kcount = lax.fori_loop(0, N // L, _scan, jnp.int32(0))

    c0 = jnp.broadcast_to(cols_s[pl.ds(0, L)][0], (L,))
    j0 = jnp.broadcast_to(jw_s[pl.ds(0, L)][0], (L,))
    for i in range(128 // L):
        cols_s[pl.ds(kcount + i * L, L)] = c0
        jw_s[pl.ds(kcount + i * L, L)] = j0

    nch = (kcount + 127) // 128
    nsub = nch * 8

    def _prefetch(r, buf):
        return pltpu.make_async_copy(v_hbm.at[wid * RPW + r], buf, sem_v)

    _prefetch(0, vrow_a).start()

    def _rowpair(g, _):
        for pp in range(2):
            r = g * 2 + pp
            buf = vrows[pp]
            _prefetch(r, buf).wait()

            @pl.when(r + 1 < RPW)
            def _():
                _prefetch(r + 1, vrows[1 - pp]).start()

            base = (wid * RPW + r) * N

            def _sub(k, __, buf=buf):
                s = k * L
                cc = cols_s[pl.ds(s, L)]
                addr_v[k // 8, pl.ds((k % 8) * L, L)] = cc + base
                w = jw_s[pl.ds(s, L)]
                val_v[k // 8, pl.ds((k % 8) * L, L)] = plsc.load_gather(
                    buf, [w])
                return 0

            lax.fori_loop(0, nsub, _sub, 0)

            for d in range(B // 128):
                @pl.when(d < nch)
                def _(d=d):
                    pltpu.async_copy(
                        val_v.at[d], out_hbm.at[addr_v.at[d]], sem_s)

            for d in range(B // 128):
                @pl.when(d < nch)
                def _():
                    pltpu.make_async_copy(
                        out_hbm.at[pl.ds(0, 128)], val_v.at[0], sem_s).wait()
        return 0

    lax.fori_loop(0, RPW // 2, _rowpair, 0)


def kernel(t, idx, v):
    t1 = t.reshape(R * N)
    v2 = v.reshape(R, B)

    out0 = pl.pallas_call(
        _copy_body,
        out_shape=jax.ShapeDtypeStruct((R * N,), jnp.float32),
        in_specs=[_HBM],
        out_specs=_HBM,
        scratch_shapes=[
            pltpu.VMEM((SLOTS, CH), jnp.float32),
            pltpu.SemaphoreType.DMA((SLOTS,)),
            pltpu.SemaphoreType.DMA((SLOTS,)),
        ],
    )(t1)

    ref = jax.new_ref(out0)
    _sc_scatter(idx, v2, ref)
    return ref[...].reshape(t.shape)


# speedup vs baseline: 2.4731x; 1.3912x over previous
import functools

import jax
import jax.numpy as jnp
from jax import lax
from jax.experimental import pallas as pl
from jax.experimental.pallas import tpu as pltpu
from jax.experimental.pallas import tpu_sc as plsc

R = 512
N = 100000
B = 4096
L = 16
NC = 2
NS = 16
NW = NC * NS
RPW = R // NW

_HBM = pl.BlockSpec(memory_space=pltpu.MemorySpace.HBM)

CH = 256000
NCHUNKS = (R * N) // CH
SLOTS = 8
LAG = 4


def _copy_body(t_ref, o_ref, buf, sem_in, sem_out):
    def _in(i, p):
        return pltpu.make_async_copy(
            t_ref.at[pl.ds(i * CH, CH)], buf.at[p], sem_in.at[p])

    def _out(i, p):
        return pltpu.make_async_copy(
            buf.at[p], o_ref.at[pl.ds(i * CH, CH)], sem_out.at[p])

    for i in range(NCHUNKS + LAG):
        if i < NCHUNKS:
            p = i % SLOTS
            if i >= SLOTS:
                _out(i - SLOTS, p).wait()
            _in(i, p).start()
        j = i - LAG
        if 0 <= j < NCHUNKS:
            pj = j % SLOTS
            _in(j, pj).wait()
            _out(j, pj).start()
    for j in range(NCHUNKS - SLOTS, NCHUNKS):
        _out(j, j % SLOTS).wait()


_mesh = plsc.VectorSubcoreMesh(
    core_axis_name="c", subcore_axis_name="s", num_cores=NC, num_subcores=NS)


@functools.partial(
    pl.kernel,
    mesh=_mesh,
    compiler_params=pltpu.CompilerParams(needs_layout_passes=False),
    scratch_types=[
        pltpu.VMEM((B,), jnp.int32),
        pltpu.VMEM((N,), jnp.int32),
        pltpu.VMEM((B,), jnp.int32),
        pltpu.VMEM((B,), jnp.float32),
        pltpu.VMEM((B,), jnp.int32),
        pltpu.VMEM((B,), jnp.int32),
        pltpu.VMEM((B,), jnp.float32),
        pltpu.VMEM((B,), jnp.float32),
        pltpu.SemaphoreType.DMA,
    ],
)
def _sc_scatter(idx_hbm, v_hbm, out_hbm,
                idx_v, m_v, winj_v, vrow_v, addr0, addr1, val0, val1, sem):
    addr_b = (addr0, addr1)
    val_b = (val0, val1)
    wid = lax.axis_index("s") * NC + lax.axis_index("c")
    pltpu.sync_copy(idx_hbm, idx_v)
    lane = lax.iota(jnp.int32, L)

    def _mb(k, _):
        idxc = idx_v[pl.ds(k * L, L)]
        jvec = k * L + lane
        plsc.store_scatter(m_v, [idxc], jvec)
        w0 = plsc.load_gather(m_v, [idxc])

        @pl.when(plsc.all_reduce_population_count(w0 != jvec)[0] > 0)
        def _():
            def _fix(i, __):
                w = plsc.load_gather(m_v, [idxc])
                plsc.store_scatter(m_v, [idxc], jvec, mask=jvec > w)
                return 0

            lax.fori_loop(0, 15, _fix, 0)

        return 0

    lax.fori_loop(0, B // L, _mb, 0)

    def _wj(k4, _):
        for u in range(4):
            k = k4 * 4 + u
            idxc = idx_v[pl.ds(k * L, L)]
            winj_v[pl.ds(k * L, L)] = plsc.load_gather(m_v, [idxc])
        return 0

    lax.fori_loop(0, B // (4 * L), _wj, 0)

    def _stage(r, p):
        rg = wid * RPW + r
        pltpu.sync_copy(v_hbm.at[rg], vrow_v)
        base = rg * N

        def _chunk(k4, _):
            for u in range(4):
                s = (k4 * 4 + u) * L
                idxc = idx_v[pl.ds(s, L)]
                addr_b[p][pl.ds(s, L)] = idxc + base
                w = winj_v[pl.ds(s, L)]
                val_b[p][pl.ds(s, L)] = plsc.load_gather(vrow_v, [w])
            return 0

        lax.fori_loop(0, B // (4 * L), _chunk, 0)

    def _fire(p):
        return pltpu.async_copy(val_b[p], out_hbm.at[addr_b[p]], sem)

    _stage(0, 0)
    h = _fire(0)
    for r in range(1, RPW):
        p = r % 2
        _stage(r, p)
        h.wait()
        h = _fire(p)
    h.wait()


def kernel(t, idx, v):
    t1 = t.reshape(R * N)
    v2 = v.reshape(R, B)

    out0 = pl.pallas_call(
        _copy_body,
        out_shape=jax.ShapeDtypeStruct((R * N,), jnp.float32),
        in_specs=[_HBM],
        out_specs=_HBM,
        scratch_shapes=[
            pltpu.VMEM((SLOTS, CH), jnp.float32),
            pltpu.SemaphoreType.DMA((SLOTS,)),
            pltpu.SemaphoreType.DMA((SLOTS,)),
        ],
    )(t1)

    ref = jax.new_ref(out0)
    _sc_scatter(idx, v2, ref)
    return ref[...].reshape(t.shape)


# speedup vs baseline: 6.3607x; 2.5720x over previous
import functools

import jax
import jax.numpy as jnp
from jax import lax
from jax.experimental import pallas as pl
from jax.experimental.pallas import tpu as pltpu
from jax.experimental.pallas import tpu_sc as plsc

R = 512
N = 100000
B = 4096
L = 16
NC = 2
NS = 16
NW = NC * NS
TCR = 32
TRPW = (R - TCR) // NW

_HBM = pl.BlockSpec(memory_space=pltpu.MemorySpace.HBM)

CH = 128000
NCHUNKS = (TCR * N) // CH
SLOTS = 8
LAG = 4


def _copy_body(t_ref, o_ref, buf, sem_in, sem_out):
    def _in(i, p):
        return pltpu.make_async_copy(
            t_ref.at[pl.ds(i * CH, CH)], buf.at[p], sem_in.at[p])

    def _out(i, p):
        return pltpu.make_async_copy(
            buf.at[p], o_ref.at[pl.ds(i * CH, CH)], sem_out.at[p])

    for i in range(NCHUNKS + LAG):
        if i < NCHUNKS:
            p = i % SLOTS
            if i >= SLOTS:
                _out(i - SLOTS, p).wait()
            _in(i, p).start()
        j = i - LAG
        if 0 <= j < NCHUNKS:
            pj = j % SLOTS
            _in(j, pj).wait()
            _out(j, pj).start()
    for j in range(max(NCHUNKS - SLOTS, 0), NCHUNKS):
        _out(j, j % SLOTS).wait()


_mesh = plsc.VectorSubcoreMesh(
    core_axis_name="c", subcore_axis_name="s", num_cores=NC, num_subcores=NS)


@functools.partial(
    pl.kernel,
    mesh=_mesh,
    compiler_params=pltpu.CompilerParams(needs_layout_passes=False),
    scratch_types=[
        pltpu.VMEM((B,), jnp.int32),
        pltpu.VMEM((N,), jnp.int32),
        pltpu.VMEM((B,), jnp.int32),
        pltpu.VMEM((B,), jnp.int32),
        pltpu.SemaphoreType.DMA,
        pltpu.SemaphoreType.DMA,
    ],
)
def _sc_scatter(idx_hbm, t_hbm, v_hbm, out_hbm,
                idx_v, m_v, winj_v, vrow_v, sem_i, sem_v):
    wid = lax.axis_index("s") * NC + lax.axis_index("c")
    pltpu.sync_copy(idx_hbm, idx_v)
    lane = lax.iota(jnp.int32, L)

    def _mb(k, _):
        idxc = idx_v[pl.ds(k * L, L)]
        jvec = k * L + lane
        plsc.store_scatter(m_v, [idxc], jvec)
        w0 = plsc.load_gather(m_v, [idxc])

        @pl.when(plsc.all_reduce_population_count(w0 != jvec)[0] > 0)
        def _():
            def _fix(i, __):
                w = plsc.load_gather(m_v, [idxc])
                plsc.store_scatter(m_v, [idxc], jvec, mask=jvec > w)
                return 0

            lax.fori_loop(0, 15, _fix, 0)

        return 0

    lax.fori_loop(0, B // L, _mb, 0)

    def _wj(k4, _):
        for u in range(4):
            k = k4 * 4 + u
            idxc = idx_v[pl.ds(k * L, L)]
            winj_v[pl.ds(k * L, L)] = plsc.load_gather(m_v, [idxc])
        return 0

    lax.fori_loop(0, B // (4 * L), _wj, 0)

    def _do_row(rg, src_off, src_ref):
        pltpu.make_async_copy(src_ref.at[pl.ds(src_off, N)],
                              m_v, sem_i).start()
        pltpu.sync_copy(v_hbm.at[pl.ds(rg * B, B)], vrow_v)
        pltpu.make_async_copy(src_ref.at[pl.ds(src_off, N)],
                              m_v, sem_i).wait()

        def _patch(k4, _):
            for u in range(4):
                s = (k4 * 4 + u) * L
                idxc = idx_v[pl.ds(s, L)]
                w = winj_v[pl.ds(s, L)]
                vals = plsc.load_gather(vrow_v, [w])
                plsc.store_scatter(m_v, [idxc], vals)
            return 0

        lax.fori_loop(0, B // (4 * L), _patch, 0)
        pltpu.sync_copy(m_v, out_hbm.at[pl.ds(rg * N, N)])

    def _trow(r, _):
        rg = TCR + wid * TRPW + r
        _do_row(rg, rg * N, t_hbm)
        return 0

    lax.fori_loop(0, TRPW, _trow, 0)

    _do_row(wid, wid * N, out_hbm)


def kernel(t, idx, v):
    ti = lax.bitcast_convert_type(t, jnp.int32).reshape(R * N)
    vi = lax.bitcast_convert_type(v, jnp.int32).reshape(R * B)

    out0 = pl.pallas_call(
        _copy_body,
        out_shape=jax.ShapeDtypeStruct((R * N,), jnp.int32),
        in_specs=[_HBM],
        out_specs=_HBM,
        scratch_shapes=[
            pltpu.VMEM((SLOTS, CH), jnp.int32),
            pltpu.SemaphoreType.DMA((SLOTS,)),
            pltpu.SemaphoreType.DMA((SLOTS,)),
        ],
    )(ti)

    ref = jax.new_ref(out0)
    _sc_scatter(idx, ti, vi, ref)
    outi = ref[...].reshape(t.shape)
    return lax.bitcast_convert_type(outi, jnp.float32)
